# baseline (speedup 1.0000x reference)
import functools

import jax
import jax.numpy as jnp
from jax import lax
from jax.experimental import pallas as pl
from jax.experimental.pallas import tpu as pltpu
from jax.experimental.pallas import tpu_sc as plsc

BATCH = 4
ROWS = 8192
D = 1024

NC = 2
NS = 16
NW = NC * NS
RPW = ROWS // NW
C = 64
NCHUNKS = RPW // C

SC_B = 3
TC_B = BATCH - SC_B

_mesh = plsc.VectorSubcoreMesh(core_axis_name="c", subcore_axis_name="s")


@functools.partial(
    pl.kernel,
    mesh=_mesh,
    out_type=jax.ShapeDtypeStruct((SC_B * ROWS, D), jnp.float32),
    scratch_types=[
        pltpu.VMEM((C, D), jnp.float32),
        pltpu.SemaphoreType.DMA,
    ],
)
def _sc_copy(w_hbm, out_hbm, buf, ssem):
    wid = lax.axis_index("c") * NS + lax.axis_index("s")
    base = wid * RPW

    for ci in range(NCHUNKS):
        r0 = base + ci * C
        pltpu.sync_copy(w_hbm.at[pl.ds(r0, C)], buf)
        for b in range(SC_B):
            pltpu.make_async_copy(
                buf, out_hbm.at[pl.ds(b * ROWS + r0, C)], ssem).start()
        for b in range(SC_B):
            pltpu.make_async_copy(
                buf, out_hbm.at[pl.ds(b * ROWS + r0, C)], ssem).wait()


R_BLK = 512


def _tc_body(w_ref, o_ref):
    o_ref[...] = jnp.broadcast_to(w_ref[...][None], (TC_B, R_BLK, D))


def kernel(input_ids, weight):
    del input_ids
    sc_part = _sc_copy(weight).reshape(SC_B, ROWS, D)
    tc_part = pl.pallas_call(
        _tc_body,
        grid=(ROWS // R_BLK,),
        in_specs=[pl.BlockSpec((R_BLK, D), lambda i: (i, 0))],
        out_specs=pl.BlockSpec((TC_B, R_BLK, D), lambda i: (0, i, 0)),
        out_shape=jax.ShapeDtypeStruct((TC_B, ROWS, D), jnp.float32),
    )(weight)
    return jnp.concatenate([sc_part, tc_part], axis=0)


# speedup vs baseline: 1.3691x; 1.3691x over previous
import functools

import jax
import jax.numpy as jnp
from jax import lax
from jax.experimental import pallas as pl
from jax.experimental.pallas import tpu as pltpu
from jax.experimental.pallas import tpu_sc as plsc

BATCH = 4
ROWS = 8192
D = 1024

NC = 2
RPC = ROWS // NC
CH = 512
NCHUNKS = RPC // CH

_mesh = plsc.ScalarSubcoreMesh(axis_name="c", num_cores=NC)


@functools.partial(
    pl.kernel,
    mesh=_mesh,
    out_type=jax.ShapeDtypeStruct((BATCH * ROWS, D), jnp.float32),
    scratch_types=[
        pltpu.VMEM_SHARED((CH, D), jnp.float32),
        pltpu.SemaphoreType.DMA,
    ],
)
def _sc_copy(w_hbm, out_hbm, buf, ssem):
    cid = lax.axis_index("c")
    base = cid * RPC

    for ci in range(NCHUNKS):
        r0 = base + ci * CH
        pltpu.sync_copy(w_hbm.at[pl.ds(r0, CH)], buf)
        for b in range(BATCH):
            pltpu.make_async_copy(
                buf, out_hbm.at[pl.ds(b * ROWS + r0, CH)], ssem).start()
        for b in range(BATCH):
            pltpu.make_async_copy(
                buf, out_hbm.at[pl.ds(b * ROWS + r0, CH)], ssem).wait()


def kernel(input_ids, weight):
    del input_ids
    out = _sc_copy(weight)
    return out.reshape(BATCH, ROWS, D)


# speedup vs baseline: 2.2926x; 1.6745x over previous
import functools

import jax
import jax.numpy as jnp
from jax import lax
from jax.experimental import pallas as pl
from jax.experimental.pallas import tpu as pltpu
from jax.experimental.pallas import tpu_sc as plsc

BATCH = 4
ROWS = 8192
D = 1024

NC = 2
NS = 16
NW = NC * NS
RPW = ROWS // NW
C = 64
NCHUNKS = RPW // C

_mesh = plsc.VectorSubcoreMesh(core_axis_name="c", subcore_axis_name="s")


@functools.partial(
    pl.kernel,
    mesh=_mesh,
    out_type=jax.ShapeDtypeStruct((BATCH * ROWS, D), jnp.float32),
    scratch_types=[
        pltpu.VMEM((C, D), jnp.float32),
        pltpu.SemaphoreType.DMA,
    ],
)
def _sc_copy(w_hbm, out_hbm, buf, ssem):
    wid = lax.axis_index("c") * NS + lax.axis_index("s")
    base = wid * RPW

    for ci in range(NCHUNKS):
        r0 = base + ci * C
        pltpu.sync_copy(w_hbm.at[pl.ds(r0, C)], buf)
        for b in range(BATCH):
            pltpu.make_async_copy(
                buf, out_hbm.at[pl.ds(b * ROWS + r0, C)], ssem).start()
        for b in range(BATCH):
            pltpu.make_async_copy(
                buf, out_hbm.at[pl.ds(b * ROWS + r0, C)], ssem).wait()


def kernel(input_ids, weight):
    del input_ids
    out = _sc_copy(weight)
    return out.reshape(BATCH, ROWS, D)
